# baseline (speedup 1.0000x reference)
import functools

import jax
import jax.numpy as jnp
from jax import lax
from jax.experimental import pallas as pl
from jax.experimental.pallas import tpu as pltpu
from jax.experimental.pallas import tpu_sc as plsc

_L = 16
_ROWS = 16
_COLS = 4096
_HALF = _COLS // 2
_HALF_CHUNKS = _HALF // _L

_mesh = plsc.VectorSubcoreMesh(core_axis_name="c", subcore_axis_name="s")


def _popcount(mb):
    p = plsc.all_reduce_population_count(mb)
    return lax.squeeze(lax.slice_in_dim(p, 0, 1), (0,))


@functools.partial(
    pl.kernel,
    out_type=jax.ShapeDtypeStruct((32, 2, _L), jnp.float32),
    mesh=_mesh,
    scratch_types=[
        pltpu.VMEM((_HALF,), jnp.float32),
        pltpu.VMEM((_COLS + _L,), jnp.float32),
        pltpu.VMEM((_COLS,), jnp.int32),
        pltpu.VMEM((2, _L), jnp.float32),
    ],
    compiler_params=pltpu.CompilerParams(needs_layout_passes=False),
)
def _sc_partials(pred_hbm, tgt_hbm, mask_hbm, out_hbm, pred_v, tgt_v, mask_v, out_v):
    c = lax.axis_index("c")
    s = lax.axis_index("s")

    pltpu.sync_copy(pred_hbm.at[s, pl.ds(c * _HALF, _HALF)], pred_v)
    pltpu.sync_copy(tgt_hbm.at[s], tgt_v.at[pl.ds(0, _COLS)])
    pltpu.sync_copy(mask_hbm.at[s], mask_v)

    def pre_body(i, acc):
        mb = mask_v[pl.ds(i * _L, _L)] != 0
        return acc + _popcount(mb)

    cnt_first = lax.fori_loop(0, _HALF_CHUNKS, pre_body, jnp.int32(0))
    off0 = jnp.where(c == 1, cnt_first, jnp.int32(0))

    def body(i, st):
        off, mis = st
        mb = mask_v[pl.ds((c * _HALF_CHUNKS + i) * _L, _L)] != 0
        pv = pred_v[pl.ds(i * _L, _L)]
        g = plsc.load_expanded(tgt_v.at[pl.ds(off, _L)], mask=mb)
        x = lax.bitcast_convert_type(pv, jnp.int32) ^ lax.bitcast_convert_type(
            g, jnp.int32
        )
        mism = (x < 0) & mb
        return (off + _popcount(mb), mis + _popcount(mism))

    off_end, mis = lax.fori_loop(
        0, _HALF_CHUNKS, body, (off0, jnp.int32(0))
    )
    cnt = off_end - off0

    lane0 = lax.iota(jnp.int32, _L) == 0
    out_v[0, :] = jnp.where(lane0, mis.astype(jnp.float32), 0.0)
    out_v[1, :] = jnp.where(lane0, cnt.astype(jnp.float32), 0.0)
    wid = s * 2 + c
    pltpu.sync_copy(out_v, out_hbm.at[wid])


def _reduce_body(p_ref, o_ref):
    mis = jnp.sum(p_ref[:, 0, :])
    cnt = jnp.sum(p_ref[:, 1, :])
    o_ref[...] = jnp.full((1, 1), 100.0 * mis / cnt, jnp.float32)


def kernel(prediction, target, mask):
    partials = _sc_partials(prediction, target, mask)
    out = pl.pallas_call(
        _reduce_body,
        out_shape=jax.ShapeDtypeStruct((1, 1), jnp.float32),
    )(partials)
    return out[0, 0]


# speedup vs baseline: 1.0565x; 1.0565x over previous
import functools

import jax
import jax.numpy as jnp
from jax import lax
from jax.experimental import pallas as pl
from jax.experimental.pallas import tpu as pltpu
from jax.experimental.pallas import tpu_sc as plsc

_L = 16
_ROWS = 16
_COLS = 4096
_HALF = _COLS // 2
_HALF_CHUNKS = _HALF // _L

_mesh = plsc.VectorSubcoreMesh(core_axis_name="c", subcore_axis_name="s")


def _popcount(mb):
    p = plsc.all_reduce_population_count(mb)
    return lax.squeeze(lax.slice_in_dim(p, 0, 1), (0,))


@functools.partial(
    pl.kernel,
    out_type=jax.ShapeDtypeStruct((32, 2, _L), jnp.float32),
    mesh=_mesh,
    scratch_types=[
        pltpu.VMEM((_HALF,), jnp.float32),
        pltpu.VMEM((_COLS + _L,), jnp.float32),
        pltpu.VMEM((_COLS,), jnp.int32),
        pltpu.VMEM((2, _L), jnp.float32),
        pltpu.SemaphoreType.DMA,
        pltpu.SemaphoreType.DMA,
        pltpu.SemaphoreType.DMA,
    ],
    compiler_params=pltpu.CompilerParams(needs_layout_passes=False),
)
def _sc_partials(pred_hbm, tgt_hbm, mask_hbm, out_hbm,
                 pred_v, tgt_v, mask_v, out_v, sem_m, sem_p, sem_t):
    c = lax.axis_index("c")
    s = lax.axis_index("s")

    cp_m = pltpu.async_copy(mask_hbm.at[s], mask_v, sem_m)
    cp_p = pltpu.async_copy(pred_hbm.at[s, pl.ds(c * _HALF, _HALF)], pred_v, sem_p)
    cp_t = pltpu.async_copy(tgt_hbm.at[s], tgt_v.at[pl.ds(0, _COLS)], sem_t)
    cp_m.wait()

    def pre_body(i, acc):
        mb = mask_v[pl.ds(i * _L, _L)] != 0
        return acc + _popcount(mb)

    cnt_first = lax.fori_loop(0, _HALF_CHUNKS, pre_body, jnp.int32(0))
    off0 = jnp.where(c == 1, cnt_first, jnp.int32(0))
    cp_p.wait()
    cp_t.wait()

    def body(i, st):
        off, mis = st
        mb = mask_v[pl.ds((c * _HALF_CHUNKS + i) * _L, _L)] != 0
        pv = pred_v[pl.ds(i * _L, _L)]
        g = plsc.load_expanded(tgt_v.at[pl.ds(off, _L)], mask=mb)
        x = lax.bitcast_convert_type(pv, jnp.int32) ^ lax.bitcast_convert_type(
            g, jnp.int32
        )
        mism = (x < 0) & mb
        return (off + _popcount(mb), mis + _popcount(mism))

    off_end, mis = lax.fori_loop(
        0, _HALF_CHUNKS, body, (off0, jnp.int32(0))
    )
    cnt = off_end - off0

    lane0 = lax.iota(jnp.int32, _L) == 0
    out_v[0, :] = jnp.where(lane0, mis.astype(jnp.float32), 0.0)
    out_v[1, :] = jnp.where(lane0, cnt.astype(jnp.float32), 0.0)
    wid = s * 2 + c
    pltpu.sync_copy(out_v, out_hbm.at[wid])


def _reduce_body(p_ref, o_ref):
    mis = jnp.sum(p_ref[:, 0, :])
    cnt = jnp.sum(p_ref[:, 1, :])
    o_ref[...] = jnp.full((1, 1), 100.0 * mis / cnt, jnp.float32)


def kernel(prediction, target, mask):
    partials = _sc_partials(prediction, target, mask)
    out = pl.pallas_call(
        _reduce_body,
        out_shape=jax.ShapeDtypeStruct((1, 1), jnp.float32),
    )(partials)
    return out[0, 0]


# speedup vs baseline: 1.0664x; 1.0094x over previous
import functools

import jax
import jax.numpy as jnp
from jax import lax
from jax.experimental import pallas as pl
from jax.experimental.pallas import tpu as pltpu
from jax.experimental.pallas import tpu_sc as plsc

_L = 16
_ROWS = 16
_COLS = 4096
_HALF = _COLS // 2
_HALF_CHUNKS = _HALF // _L

_mesh = plsc.VectorSubcoreMesh(core_axis_name="c", subcore_axis_name="s")


def _popcount(mb):
    p = plsc.all_reduce_population_count(mb)
    return lax.squeeze(lax.slice_in_dim(p, 0, 1), (0,))


@functools.partial(
    pl.kernel,
    out_type=jax.ShapeDtypeStruct((8, 128), jnp.float32),
    mesh=_mesh,
    scratch_types=[
        pltpu.VMEM((_HALF,), jnp.float32),
        pltpu.VMEM((_COLS + _L,), jnp.float32),
        pltpu.VMEM((_COLS,), jnp.int32),
        pltpu.VMEM((2 * _L,), jnp.float32),
        pltpu.SemaphoreType.DMA,
        pltpu.SemaphoreType.DMA,
        pltpu.SemaphoreType.DMA,
    ],
    compiler_params=pltpu.CompilerParams(needs_layout_passes=False),
)
def _sc_partials(pred_hbm, tgt_hbm, mask_hbm, out_hbm,
                 pred_v, tgt_v, mask_v, out_v, sem_m, sem_p, sem_t):
    c = lax.axis_index("c")
    s = lax.axis_index("s")

    cp_m = pltpu.async_copy(mask_hbm.at[s], mask_v, sem_m)
    cp_p = pltpu.async_copy(pred_hbm.at[s, pl.ds(c * _HALF, _HALF)], pred_v, sem_p)
    cp_t = pltpu.async_copy(tgt_hbm.at[s], tgt_v.at[pl.ds(0, _COLS)], sem_t)
    cp_m.wait()

    def pre_body(i, acc):
        mb = mask_v[pl.ds(i * _L, _L)] != 0
        return acc + _popcount(mb)

    cnt_first = lax.fori_loop(0, _HALF_CHUNKS, pre_body, jnp.int32(0))
    off0 = jnp.where(c == 1, cnt_first, jnp.int32(0))
    cp_p.wait()
    cp_t.wait()

    def chunk(idx, off, mis):
        mb = mask_v[pl.ds(idx * _L, _L)] != 0
        pv = pred_v[pl.ds((idx - c * _HALF_CHUNKS) * _L, _L)]
        g = plsc.load_expanded(tgt_v.at[pl.ds(off, _L)], mask=mb)
        x = lax.bitcast_convert_type(pv, jnp.int32) ^ lax.bitcast_convert_type(
            g, jnp.int32
        )
        mism = (x < 0) & mb
        return off + _popcount(mb), mis + _popcount(mism)

    def body(i, st):
        off, mis = st
        base = c * _HALF_CHUNKS + i * 2
        off, mis = chunk(base, off, mis)
        off, mis = chunk(base + 1, off, mis)
        return (off, mis)

    off_end, mis = lax.fori_loop(
        0, _HALF_CHUNKS // 2, body, (off0, jnp.int32(0))
    )
    cnt = off_end - off0

    lane0 = lax.iota(jnp.int32, _L) == 0
    out_v[pl.ds(0, _L)] = jnp.where(lane0, mis.astype(jnp.float32), 0.0)
    out_v[pl.ds(_L, _L)] = jnp.where(lane0, cnt.astype(jnp.float32), 0.0)
    wid = s * 2 + c
    row = lax.shift_right_logical(wid, 2)
    col = (wid & 3) * (2 * _L)
    pltpu.sync_copy(out_v, out_hbm.at[row, pl.ds(col, 2 * _L)])


def _reduce_body(p_ref, o_ref):
    x = p_ref[...]
    is_mis = (lax.broadcasted_iota(jnp.int32, (8, 128), 1) & _L) == 0
    mis = jnp.sum(jnp.where(is_mis, x, 0.0))
    cnt = jnp.sum(jnp.where(is_mis, 0.0, x))
    o_ref[...] = jnp.full((1, 1), 100.0 * mis / cnt, jnp.float32)


def kernel(prediction, target, mask):
    partials = _sc_partials(prediction, target, mask)
    out = pl.pallas_call(
        _reduce_body,
        out_shape=jax.ShapeDtypeStruct((1, 1), jnp.float32),
    )(partials)
    return out[0, 0]


# speedup vs baseline: 1.0815x; 1.0142x over previous
import functools

import jax
import jax.numpy as jnp
from jax import lax
from jax.experimental import pallas as pl
from jax.experimental.pallas import tpu as pltpu
from jax.experimental.pallas import tpu_sc as plsc

_L = 16
_ROWS = 16
_COLS = 4096
_HALF = _COLS // 2
_HALF_CHUNKS = _HALF // _L

_mesh = plsc.VectorSubcoreMesh(core_axis_name="c", subcore_axis_name="s")


def _popcount(mb):
    p = plsc.all_reduce_population_count(mb)
    return lax.squeeze(lax.slice_in_dim(p, 0, 1), (0,))


@functools.partial(
    pl.kernel,
    out_type=jax.ShapeDtypeStruct((8, 128), jnp.float32),
    mesh=_mesh,
    scratch_types=[
        pltpu.VMEM((_HALF,), jnp.float32),
        pltpu.VMEM((_COLS + _L,), jnp.float32),
        pltpu.VMEM((_COLS,), jnp.int32),
        pltpu.VMEM((2 * _L,), jnp.float32),
        pltpu.SemaphoreType.DMA,
        pltpu.SemaphoreType.DMA,
        pltpu.SemaphoreType.DMA,
    ],
    compiler_params=pltpu.CompilerParams(needs_layout_passes=False),
)
def _sc_partials(pred_hbm, tgt_hbm, mask_hbm, out_hbm,
                 pred_v, tgt_v, mask_v, out_v, sem_m, sem_p, sem_t):
    c = lax.axis_index("c")
    s = lax.axis_index("s")

    cp_m = pltpu.async_copy(mask_hbm.at[s], mask_v, sem_m)
    cp_p = pltpu.async_copy(pred_hbm.at[s, pl.ds(c * _HALF, _HALF)], pred_v, sem_p)
    cp_t = pltpu.async_copy(tgt_hbm.at[s], tgt_v.at[pl.ds(0, _COLS)], sem_t)
    cp_m.wait()

    def pre_body(i, acc):
        mb0 = mask_v[pl.ds((2 * i) * _L, _L)] != 0
        mb1 = mask_v[pl.ds((2 * i + 1) * _L, _L)] != 0
        return acc + _popcount(mb0) + _popcount(mb1)

    cnt_first = lax.fori_loop(0, _HALF_CHUNKS // 2, pre_body, jnp.int32(0))
    off0 = jnp.where(c == 1, cnt_first, jnp.int32(0))
    cp_p.wait()
    cp_t.wait()

    def chunk(idx, off, mis):
        mb = mask_v[pl.ds(idx * _L, _L)] != 0
        pv = pred_v[pl.ds((idx - c * _HALF_CHUNKS) * _L, _L)]
        g = plsc.load_expanded(tgt_v.at[pl.ds(off, _L)], mask=mb)
        x = lax.bitcast_convert_type(pv, jnp.int32) ^ lax.bitcast_convert_type(
            g, jnp.int32
        )
        mism = (x < 0) & mb
        return off + _popcount(mb), mis + _popcount(mism)

    def body(i, st):
        off, mis = st
        base = c * _HALF_CHUNKS + i * 2
        off, mis = chunk(base, off, mis)
        off, mis = chunk(base + 1, off, mis)
        return (off, mis)

    off_end, mis = lax.fori_loop(
        0, _HALF_CHUNKS // 2, body, (off0, jnp.int32(0))
    )
    cnt = off_end - off0

    lane0 = lax.iota(jnp.int32, _L) == 0
    out_v[pl.ds(0, _L)] = jnp.where(lane0, mis.astype(jnp.float32), 0.0)
    out_v[pl.ds(_L, _L)] = jnp.where(lane0, cnt.astype(jnp.float32), 0.0)
    wid = s * 2 + c
    row = lax.shift_right_logical(wid, 2)
    col = (wid & 3) * (2 * _L)
    pltpu.sync_copy(out_v, out_hbm.at[row, pl.ds(col, 2 * _L)])


def _reduce_body(p_ref, o_ref):
    x = p_ref[...]
    is_mis = (lax.broadcasted_iota(jnp.int32, (8, 128), 1) & _L) == 0
    mis = jnp.sum(jnp.where(is_mis, x, 0.0))
    cnt = jnp.sum(jnp.where(is_mis, 0.0, x))
    o_ref[...] = jnp.full((1, 1), 100.0 * mis / cnt, jnp.float32)


def kernel(prediction, target, mask):
    partials = _sc_partials(prediction, target, mask)
    out = pl.pallas_call(
        _reduce_body,
        out_shape=jax.ShapeDtypeStruct((1, 1), jnp.float32),
    )(partials)
    return out[0, 0]


# speedup vs baseline: 1.0847x; 1.0029x over previous
import functools

import jax
import jax.numpy as jnp
from jax import lax
from jax.experimental import pallas as pl
from jax.experimental.pallas import tpu as pltpu
from jax.experimental.pallas import tpu_sc as plsc

_L = 16
_ROWS = 16
_COLS = 4096
_HALF = _COLS // 2
_HALF_CHUNKS = _HALF // _L

_mesh = plsc.VectorSubcoreMesh(core_axis_name="c", subcore_axis_name="s")


def _popcount(mb):
    p = plsc.all_reduce_population_count(mb)
    return lax.squeeze(lax.slice_in_dim(p, 0, 1), (0,))


@functools.partial(
    pl.kernel,
    out_type=jax.ShapeDtypeStruct((8, 128), jnp.float32),
    mesh=_mesh,
    scratch_types=[
        pltpu.VMEM((_HALF,), jnp.float32),
        pltpu.VMEM((_COLS + _L,), jnp.float32),
        pltpu.VMEM((_COLS,), jnp.int32),
        pltpu.VMEM((2 * _L,), jnp.float32),
        pltpu.SemaphoreType.DMA,
        pltpu.SemaphoreType.DMA,
        pltpu.SemaphoreType.DMA,
    ],
    compiler_params=pltpu.CompilerParams(needs_layout_passes=False),
)
def _sc_partials(pred_hbm, tgt_hbm, mask_hbm, out_hbm,
                 pred_v, tgt_v, mask_v, out_v, sem_m, sem_p, sem_t):
    c = lax.axis_index("c")
    s = lax.axis_index("s")

    cp_m = pltpu.async_copy(mask_hbm.at[s], mask_v, sem_m)
    cp_p = pltpu.async_copy(pred_hbm.at[s, pl.ds(c * _HALF, _HALF)], pred_v, sem_p)
    cp_t = pltpu.async_copy(tgt_hbm.at[s], tgt_v.at[pl.ds(0, _COLS)], sem_t)
    cp_m.wait()

    def pre_body(i, acc):
        mb0 = mask_v[pl.ds((2 * i) * _L, _L)] != 0
        mb1 = mask_v[pl.ds((2 * i + 1) * _L, _L)] != 0
        return acc + _popcount(mb0) + _popcount(mb1)

    cnt_first = lax.fori_loop(0, _HALF_CHUNKS // 2, pre_body, jnp.int32(0))
    off0 = jnp.where(c == 1, cnt_first, jnp.int32(0))
    cp_p.wait()
    cp_t.wait()

    def chunk(idx, off, mis):
        mb = mask_v[pl.ds(idx * _L, _L)] != 0
        pv = pred_v[pl.ds((idx - c * _HALF_CHUNKS) * _L, _L)]
        g = plsc.load_expanded(tgt_v.at[pl.ds(off, _L)], mask=mb)
        x = lax.bitcast_convert_type(pv, jnp.int32) ^ lax.bitcast_convert_type(
            g, jnp.int32
        )
        mism = (x < 0) & mb
        return off + _popcount(mb), mis + _popcount(mism)

    def body(i, st):
        off, mis = st
        base = c * _HALF_CHUNKS + i * 4
        off, mis = chunk(base, off, mis)
        off, mis = chunk(base + 1, off, mis)
        off, mis = chunk(base + 2, off, mis)
        off, mis = chunk(base + 3, off, mis)
        return (off, mis)

    off_end, mis = lax.fori_loop(
        0, _HALF_CHUNKS // 4, body, (off0, jnp.int32(0))
    )
    cnt = off_end - off0

    lane0 = lax.iota(jnp.int32, _L) == 0
    out_v[pl.ds(0, _L)] = jnp.where(lane0, mis.astype(jnp.float32), 0.0)
    out_v[pl.ds(_L, _L)] = jnp.where(lane0, cnt.astype(jnp.float32), 0.0)
    wid = s * 2 + c
    row = lax.shift_right_logical(wid, 2)
    col = (wid & 3) * (2 * _L)
    pltpu.sync_copy(out_v, out_hbm.at[row, pl.ds(col, 2 * _L)])


def _reduce_body(p_ref, o_ref):
    x = p_ref[...]
    is_mis = (lax.broadcasted_iota(jnp.int32, (8, 128), 1) & _L) == 0
    mis = jnp.sum(jnp.where(is_mis, x, 0.0))
    cnt = jnp.sum(jnp.where(is_mis, 0.0, x))
    o_ref[...] = jnp.full((1, 1), 100.0 * mis / cnt, jnp.float32)


def kernel(prediction, target, mask):
    partials = _sc_partials(prediction, target, mask)
    out = pl.pallas_call(
        _reduce_body,
        out_shape=jax.ShapeDtypeStruct((1, 1), jnp.float32),
    )(partials)
    return out[0, 0]
